# Initial kernel scaffold; baseline (speedup 1.0000x reference)
import jax
import jax.numpy as jnp
from jax.experimental import pallas as pl


def kernel(synonyms, antonyms, hypernyms, meronyms, emb_table, vocab_freq, syn_proj_w, hypn_proj_w, mern_proj_w, hypn_rel_w, hypn_rel_b, mern_rel_w, mern_rel_b):
    raise NotImplementedError("write your pallas kernel here")


# speedup vs baseline: 7.4441x; 7.4441x over previous
import functools

import jax
import jax.numpy as jnp
from jax import lax
from jax.experimental import pallas as pl
from jax.experimental.pallas import tpu as pltpu
from jax.experimental.pallas import tpu_sc as plsc

V = 1000000
VP = 1 << 20
EMB = 64
WN = 64
B = 16384
NNEG = 10
NQ = 3 * B * NNEG
NROW = VP // 16
EPS = 1e-6

NW = 32
QPW = NQ // NW
CQ = 1536
NCH = QPW // CQ
NPAIR = 8 * B
NGATH = NPAIR + NQ
GPW = NGATH // NW


def _stage1_body(f_ref, u_ref, p_ref, c_ref, r_ref, carry_ref):
    i = pl.program_id(0)

    @pl.when(i == 0)
    def _():
        carry_ref[0] = 0.0

    x = f_ref[0]
    g = jnp.where(x > 0.0, jnp.exp(0.75 * jnp.log(jnp.where(x > 0.0, x, 1.0))), 0.0)

    ri = lax.broadcasted_iota(jnp.int32, (128, 128), 0)
    ci = lax.broadcasted_iota(jnp.int32, (128, 128), 1)
    upper = (ri <= ci).astype(jnp.float32)
    lstrict = (ri > ci).astype(jnp.float32)

    pref = lax.dot_general(g, upper, (((1,), (0,)), ((), ())),
                           precision=lax.Precision.HIGHEST,
                           preferred_element_type=jnp.float32)
    rowsum = jnp.sum(g, axis=1, keepdims=True)
    roff = lax.dot_general(lstrict, rowsum, (((1,), (0,)), ((), ())),
                           precision=lax.Precision.HIGHEST,
                           preferred_element_type=jnp.float32)
    carry = carry_ref[0]
    p = pref + roff + carry
    p_ref[0] = p

    r8 = lax.broadcasted_iota(jnp.int32, (128, 8), 0)
    c8 = lax.broadcasted_iota(jnp.int32, (128, 8), 1)
    esel = (r8 == 16 * c8 + 15).astype(jnp.float32)
    c_ref[0] = lax.dot_general(p, esel, (((1,), (0,)), ((), ())),
                               precision=lax.Precision.HIGHEST,
                               preferred_element_type=jnp.float32)

    carry_ref[0] = carry + jnp.sum(g)

    @pl.when(i == 63)
    def _():
        t = carry_ref[0]
        r_ref[...] = t * (1.0 - u_ref[...])


def _build_stage1(interpret=False):
    return pl.pallas_call(
        _stage1_body,
        grid=(64,),
        in_specs=[
            pl.BlockSpec((1, 128, 128), lambda i: (i, 0, 0)),
            pl.BlockSpec((3840, 128), lambda i: (0, 0)),
        ],
        out_specs=[
            pl.BlockSpec((1, 128, 128), lambda i: (i, 0, 0)),
            pl.BlockSpec((1, 128, 8), lambda i: (i, 0, 0)),
            pl.BlockSpec((3840, 128), lambda i: (0, 0)),
        ],
        out_shape=[
            jax.ShapeDtypeStruct((64, 128, 128), jnp.float32),
            jax.ShapeDtypeStruct((64, 128, 8), jnp.float32),
            jax.ShapeDtypeStruct((3840, 128), jnp.float32),
        ],
        scratch_shapes=[pltpu.SMEM((1,), jnp.float32)],
        interpret=interpret,
    )


_stage1 = _build_stage1()


def _stage2_body(p2d_hbm, c_hbm, r_hbm, out_hbm, cv, qv, rowv2, rowsbuf, outv, sem):
    wid = lax.axis_index("s") * 2 + lax.axis_index("c")
    base = wid * QPW
    pltpu.sync_copy(c_hbm, cv)

    def chunk_body(ch, _):
        qoff = base + ch * CQ
        pltpu.sync_copy(r_hbm.at[pl.ds(qoff, CQ)], qv)

        def coarse_body(j2, _):
            for k in range(8):
                q = qv[pl.ds(j2 * 128 + k * 16, 16)]
                lo = jnp.zeros((16,), jnp.int32)
                for st in range(16):
                    h = 1 << (15 - st)
                    probe = lo + (h - 1)
                    val = plsc.load_gather(cv, [probe])
                    lo = jnp.where(val < q, lo + h, lo)
                rowv2[j2, pl.ds(k * 16, 16)] = lo
            return 0

        lax.fori_loop(0, 12, coarse_body, 0)

        cps = [
            pltpu.async_copy(p2d_hbm.at[rowv2.at[j]], rowsbuf.at[pl.ds(j * 128, 128)], sem)
            for j in range(12)
        ]
        for cp in cps:
            cp.wait()

        def refine_body(j2, _):
            for k in range(8):
                off = j2 * 128 + k * 16
                q = qv[pl.ds(off, 16)]
                lo = rowv2[j2, pl.ds(k * 16, 16)]
                lanesel = lax.iota(jnp.int32, 16) + off
                cnt = jnp.zeros((16,), jnp.int32)
                for col in range(16):
                    vals = plsc.load_gather(
                        rowsbuf, [lanesel, jnp.full((16,), col, jnp.int32)])
                    cnt = cnt + (vals < q).astype(jnp.int32)
                idx = jnp.minimum(lo * 16 + cnt, V - 1)
                outv[pl.ds(off, 16)] = idx
            return 0

        lax.fori_loop(0, 12, refine_body, 0)
        pltpu.sync_copy(outv, out_hbm.at[pl.ds(qoff, CQ)])
        return 0

    lax.fori_loop(0, NCH, chunk_body, 0)


@functools.cache
def _get_stage2():
    return pl.kernel(
        _stage2_body,
        out_type=jax.ShapeDtypeStruct((NQ,), jnp.int32),
        mesh=plsc.VectorSubcoreMesh(core_axis_name="c", subcore_axis_name="s",
                                    num_cores=2, num_subcores=16),
        compiler_params=pltpu.CompilerParams(needs_layout_passes=False, use_tc_tiling_on_sc=False),
        scratch_types=[
            pltpu.VMEM((NROW,), jnp.float32),
            pltpu.VMEM((CQ,), jnp.float32),
            pltpu.VMEM((12, 128), jnp.int32),
            pltpu.VMEM((CQ, 16), jnp.float32),
            pltpu.VMEM((CQ,), jnp.int32),
            pltpu.SemaphoreType.DMA,
        ],
    )


def _stage3_body(idx3_hbm, emb_hbm, out_hbm, idxv, b0, b1, b2, b3,
                 g0, g1, g2, g3, s0, s1, s2, s3):
    bufs = (b0, b1, b2, b3)
    gsems = (g0, g1, g2, g3)
    ssems = (s0, s1, s2, s3)
    wid = lax.axis_index("s") * 2 + lax.axis_index("c")
    base = wid * GPW
    pltpu.sync_copy(idx3_hbm.at[wid], idxv)

    nj = GPW // 128
    gcp = [None] * nj
    scp = [None] * nj
    for j in range(nj):
        b = j % 4
        if j >= 4:
            scp[j - 4].wait()
        gcp[j] = pltpu.async_copy(emb_hbm.at[idxv.at[j]], bufs[b], gsems[b])
        if j >= 1:
            gcp[j - 1].wait()
            scp[j - 1] = pltpu.async_copy(
                bufs[(j - 1) % 4], out_hbm.at[pl.ds(base + (j - 1) * 128, 128)],
                ssems[(j - 1) % 4])
    gcp[nj - 1].wait()
    scp[nj - 1] = pltpu.async_copy(
        bufs[(nj - 1) % 4], out_hbm.at[pl.ds(base + (nj - 1) * 128, 128)],
        ssems[(nj - 1) % 4])
    for j in range(nj - 4, nj):
        scp[j].wait()


@functools.cache
def _get_stage3():
    return pl.kernel(
        _stage3_body,
        out_type=jax.ShapeDtypeStruct((NGATH, EMB), jnp.float32),
        mesh=plsc.VectorSubcoreMesh(core_axis_name="c", subcore_axis_name="s",
                                    num_cores=2, num_subcores=16),
        compiler_params=pltpu.CompilerParams(needs_layout_passes=False, use_tc_tiling_on_sc=False),
        scratch_types=(
            [pltpu.VMEM((GPW // 128, 128), jnp.int32)]
            + [pltpu.VMEM((128, EMB), jnp.float32)] * 4
            + [pltpu.SemaphoreType.DMA] * 8
        ),
    )


def _mm(x, w):
    return lax.dot_general(x, w, (((1,), (1,)), ((), ())),
                           precision=lax.Precision.HIGHEST,
                           preferred_element_type=jnp.float32)


def _pd(a, b):
    return jnp.sqrt(jnp.sum((a - b + EPS) ** 2, axis=1, keepdims=True))


def _stage4_body(*refs):
    (s0, s1, a0, a1, h0, h1, m0, m1) = refs[0:8]
    nsyn = refs[8:18]
    nhyp = refs[18:28]
    nmer = refs[28:38]
    (msk_s, msk_a, msk_h, msk_m) = refs[38:42]
    (wsyn, whyp, wmer, rh, bh, rm, bm) = refs[42:49]
    out_ref = refs[49]
    acc = refs[50]

    i = pl.program_id(0)

    @pl.when(i == 0)
    def _():
        for k in range(8):
            acc[k] = 0.0

    ws = wsyn[...]
    wh = whyp[...]
    wm = wmer[...]
    mh = lax.dot_general(rh[...], wh, (((1,), (0,)), ((), ())),
                         precision=lax.Precision.HIGHEST,
                         preferred_element_type=jnp.float32)
    mm_ = lax.dot_general(rm[...], wm, (((1,), (0,)), ((), ())),
                          precision=lax.Precision.HIGHEST,
                          preferred_element_type=jnp.float32)
    bhv = bh[...]
    bmv = bm[...]

    fm_s = 1.0 - (msk_s[...] == 0).astype(jnp.float32)
    fm_a = 1.0 - (msk_a[...] == 0).astype(jnp.float32)
    fm_h = 1.0 - (msk_h[...] == 0).astype(jnp.float32)
    fm_m = 1.0 - (msk_m[...] == 0).astype(jnp.float32)

    e1 = _mm(s0[...], ws)
    e2 = _mm(s1[...], ws)
    an = jnp.zeros_like(e1[:, :1])
    af = jnp.zeros_like(an)
    for j in range(NNEG):
        en = _mm(nsyn[j][...], ws)
        dn = _pd(e1, en)
        an = an + jnp.maximum(0.1 - dn, 0.0)
        af = af + jnp.maximum(dn - 1.5, 0.0)
    t_syn = (_pd(e1, e2) + an / NNEG + af / NNEG) * fm_s
    acc[0] = acc[0] + jnp.sum(t_syn)
    acc[1] = acc[1] + jnp.sum(fm_s)

    aa1 = _mm(a0[...], ws)
    aa2 = _mm(a1[...], ws)
    t_ant = jnp.maximum(1.0 - _pd(aa1, aa2), 0.0) * fm_a
    acc[2] = acc[2] + jnp.sum(t_ant)
    acc[3] = acc[3] + jnp.sum(fm_a)

    hh1 = _mm(h0[...], mh) + bhv
    hh2 = _mm(h1[...], wh)
    hn_acc = jnp.zeros_like(an)
    for j in range(NNEG):
        hn = _mm(nhyp[j][...], mh) + bhv
        hdn = _pd(hh2, hn)
        hn_acc = hn_acc + jnp.maximum(0.1 - hdn, 0.0)
    t_hyp = (_pd(hh1, hh2) + 3.0 * (hn_acc / NNEG)) * fm_h
    acc[4] = acc[4] + jnp.sum(t_hyp)
    acc[5] = acc[5] + jnp.sum(fm_h)

    mm1 = _mm(m0[...], mm_) + bmv
    mm2 = _mm(m1[...], wm)
    mn_acc = jnp.zeros_like(an)
    for j in range(NNEG):
        mn = _mm(nmer[j][...], mm_) + bmv
        mdn = _pd(mm2, mn)
        mn_acc = mn_acc + jnp.maximum(0.1 - mdn, 0.0)
    t_mer = (_pd(mm1, mm2) + mn_acc / NNEG) * fm_m
    acc[6] = acc[6] + jnp.sum(t_mer)
    acc[7] = acc[7] + jnp.sum(fm_m)

    @pl.when(i == pl.num_programs(0) - 1)
    def _():
        l0 = acc[0] / jnp.maximum(acc[1], 1.0)
        l1 = acc[2] / jnp.maximum(acc[3], 1.0)
        l2 = acc[4] / jnp.maximum(acc[5], 1.0)
        l3 = acc[6] / jnp.maximum(acc[7], 1.0)
        lane = lax.broadcasted_iota(jnp.int32, (1, 128), 1)
        outv = (jnp.where(lane == 0, l0, 0.0) + jnp.where(lane == 1, l1, 0.0)
                + jnp.where(lane == 2, l2, 0.0) + jnp.where(lane == 3, l3, 0.0))
        out_ref[...] = outv.astype(jnp.float32)


def _build_stage4(interpret=False):
    bs = 512
    grid = (B // bs,)
    gspec = lambda c: pl.BlockSpec((bs, EMB), lambda i, c=c: (c * (B // bs) + i, 0))
    in_specs = []
    for c in range(8):
        in_specs.append(gspec(c))
    for r in range(3):
        for j in range(NNEG):
            off = (NPAIR + r * B * NNEG + j * B) // bs
            in_specs.append(pl.BlockSpec((bs, EMB), lambda i, off=off: (off + i, 0)))
    for _ in range(4):
        in_specs.append(pl.BlockSpec((bs, 1), lambda i: (i, 0)))
    for _ in range(3):
        in_specs.append(pl.BlockSpec((WN, EMB), lambda i: (0, 0)))
    in_specs.append(pl.BlockSpec((WN, WN), lambda i: (0, 0)))
    in_specs.append(pl.BlockSpec((1, WN), lambda i: (0, 0)))
    in_specs.append(pl.BlockSpec((WN, WN), lambda i: (0, 0)))
    in_specs.append(pl.BlockSpec((1, WN), lambda i: (0, 0)))
    return pl.pallas_call(
        _stage4_body,
        grid=grid,
        in_specs=in_specs,
        out_specs=pl.BlockSpec((1, 128), lambda i: (0, 0)),
        out_shape=jax.ShapeDtypeStruct((1, 128), jnp.float32),
        scratch_shapes=[pltpu.SMEM((8,), jnp.float32)],
        compiler_params=pltpu.CompilerParams(vmem_limit_bytes=100 * 1024 * 1024),
        interpret=interpret,
    )


_stage4 = _build_stage4()


def kernel(synonyms, antonyms, hypernyms, meronyms, emb_table, vocab_freq,
           syn_proj_w, hypn_proj_w, mern_proj_w,
           hypn_rel_w, hypn_rel_b, mern_rel_w, mern_rel_b):
    skey = jax.random.key(42)
    us = [jax.random.uniform(jax.random.fold_in(skey, i), (B, NNEG), jnp.float32)
          for i in (1, 2, 3)]
    u_all = jnp.concatenate([u.T.reshape(-1) for u in us]).reshape(3840, 128)

    f_pad = jnp.concatenate(
        [vocab_freq, jnp.zeros((VP - V,), jnp.float32)]).reshape(64, 128, 128)

    p3, c3, r2 = _stage1(f_pad, u_all)
    p2d = p3.reshape(NROW, 16)
    c1 = c3.reshape(NROW)
    r1 = r2.reshape(NQ)

    nidx = _get_stage2()(p2d, c1, r1)

    pairs = jnp.stack([synonyms, antonyms, hypernyms, meronyms])
    pairs = pairs.transpose(0, 2, 1).reshape(-1).astype(jnp.int32)
    idx3 = jnp.concatenate([pairs, nidx]).reshape(NW, GPW // 128, 128)

    g = _get_stage3()(idx3, emb_table)

    gargs = [g] * 38
    margs = [synonyms[:, :1], antonyms[:, :1], hypernyms[:, :1], meronyms[:, :1]]
    wargs = [syn_proj_w, hypn_proj_w, mern_proj_w,
             hypn_rel_w, hypn_rel_b.reshape(1, WN),
             mern_rel_w, mern_rel_b.reshape(1, WN)]
    out2d = _stage4(*gargs, *margs, *wargs)
    return out2d[0, :4]


# speedup vs baseline: 8.2948x; 1.1143x over previous
import functools

import jax
import jax.numpy as jnp
from jax import lax
from jax.experimental import pallas as pl
from jax.experimental.pallas import tpu as pltpu
from jax.experimental.pallas import tpu_sc as plsc

V = 1000000
VP = 1 << 20
EMB = 64
WN = 64
B = 16384
NNEG = 10
NQ = 3 * B * NNEG
NROW = VP // 16
EPS = 1e-6

NROW32 = VP // 32
NW = 32
QPW = NQ // NW
CQ = 1536
NCH = QPW // CQ
NPAIR = 8 * B
NGATH = NPAIR + NQ
GPW = NGATH // NW


def _stage1_body(f_ref, u_ref, p_ref, c_ref, r_ref, carry_ref):
    i = pl.program_id(0)

    @pl.when(i == 0)
    def _():
        carry_ref[0] = 0.0

    x = f_ref[0]
    g = jnp.where(x > 0.0, jnp.exp(0.75 * jnp.log(jnp.where(x > 0.0, x, 1.0))), 0.0)

    ri = lax.broadcasted_iota(jnp.int32, (128, 128), 0)
    ci = lax.broadcasted_iota(jnp.int32, (128, 128), 1)
    upper = (ri <= ci).astype(jnp.float32)
    lstrict = (ri > ci).astype(jnp.float32)

    pref = lax.dot_general(g, upper, (((1,), (0,)), ((), ())),
                           precision=lax.Precision.HIGHEST,
                           preferred_element_type=jnp.float32)
    rowsum = jnp.sum(g, axis=1, keepdims=True)
    roff = lax.dot_general(lstrict, rowsum, (((1,), (0,)), ((), ())),
                           precision=lax.Precision.HIGHEST,
                           preferred_element_type=jnp.float32)
    carry = carry_ref[0]
    p = pref + roff + carry
    p_ref[0] = p

    r8 = lax.broadcasted_iota(jnp.int32, (128, 4), 0)
    c8 = lax.broadcasted_iota(jnp.int32, (128, 4), 1)
    esel = (r8 == 32 * c8 + 31).astype(jnp.float32)
    c_ref[0] = lax.dot_general(p, esel, (((1,), (0,)), ((), ())),
                               precision=lax.Precision.HIGHEST,
                               preferred_element_type=jnp.float32)

    carry_ref[0] = carry + jnp.sum(g)

    @pl.when(i == 63)
    def _():
        t = carry_ref[0]
        r_ref[...] = t * (1.0 - u_ref[...])


def _build_stage1(interpret=False):
    return pl.pallas_call(
        _stage1_body,
        grid=(64,),
        in_specs=[
            pl.BlockSpec((1, 128, 128), lambda i: (i, 0, 0)),
            pl.BlockSpec((3840, 128), lambda i: (0, 0)),
        ],
        out_specs=[
            pl.BlockSpec((1, 128, 128), lambda i: (i, 0, 0)),
            pl.BlockSpec((1, 128, 4), lambda i: (i, 0, 0)),
            pl.BlockSpec((3840, 128), lambda i: (0, 0)),
        ],
        out_shape=[
            jax.ShapeDtypeStruct((64, 128, 128), jnp.float32),
            jax.ShapeDtypeStruct((64, 128, 4), jnp.float32),
            jax.ShapeDtypeStruct((3840, 128), jnp.float32),
        ],
        scratch_shapes=[pltpu.SMEM((1,), jnp.float32)],
        interpret=interpret,
    )


_stage1 = _build_stage1()


def _stage2_body(p2d_hbm, c_hbm, r_hbm, out_hbm, cv, qv, rowv2, rowsbuf, outv, sem):
    wid = lax.axis_index("s") * 2 + lax.axis_index("c")
    base = wid * QPW
    pltpu.sync_copy(c_hbm, cv)

    def chunk_body(ch, _):
        qoff = base + ch * CQ
        pltpu.sync_copy(r_hbm.at[pl.ds(qoff, CQ)], qv)

        def coarse_body(j2, _):
            for k in range(8):
                q = qv[pl.ds(j2 * 128 + k * 16, 16)]
                lo = jnp.zeros((16,), jnp.int32)
                for st in range(15):
                    h = 1 << (14 - st)
                    probe = lo + (h - 1)
                    val = plsc.load_gather(cv, [probe])
                    lo = jnp.where(val < q, lo + h, lo)
                rowv2[j2, pl.ds(k * 16, 16)] = lo
            return 0

        lax.fori_loop(0, 12, coarse_body, 0)

        cps = [
            pltpu.async_copy(p2d_hbm.at[rowv2.at[j]], rowsbuf.at[pl.ds(j * 128, 128)], sem)
            for j in range(12)
        ]
        for cp in cps:
            cp.wait()

        def refine_body(j2, _):
            for k in range(8):
                off = j2 * 128 + k * 16
                q = qv[pl.ds(off, 16)]
                lo = rowv2[j2, pl.ds(k * 16, 16)]
                lanesel = lax.iota(jnp.int32, 16) + off
                pos = jnp.zeros((16,), jnp.int32)
                for st in range(5):
                    h = 1 << (4 - st)
                    probe = pos + (h - 1)
                    vals = plsc.load_gather(rowsbuf, [lanesel, probe])
                    pos = jnp.where(vals < q, pos + h, pos)
                idx = jnp.minimum(lo * 32 + pos, V - 1)
                outv[pl.ds(off, 16)] = idx
            return 0

        lax.fori_loop(0, 12, refine_body, 0)
        pltpu.sync_copy(outv, out_hbm.at[pl.ds(qoff, CQ)])
        return 0

    lax.fori_loop(0, NCH, chunk_body, 0)


@functools.cache
def _get_stage2():
    return pl.kernel(
        _stage2_body,
        out_type=jax.ShapeDtypeStruct((NQ,), jnp.int32),
        mesh=plsc.VectorSubcoreMesh(core_axis_name="c", subcore_axis_name="s",
                                    num_cores=2, num_subcores=16),
        compiler_params=pltpu.CompilerParams(needs_layout_passes=False, use_tc_tiling_on_sc=False),
        scratch_types=[
            pltpu.VMEM((NROW32,), jnp.float32),
            pltpu.VMEM((CQ,), jnp.float32),
            pltpu.VMEM((12, 128), jnp.int32),
            pltpu.VMEM((CQ, 32), jnp.float32),
            pltpu.VMEM((CQ,), jnp.int32),
            pltpu.SemaphoreType.DMA,
        ],
    )


def _stage3_body(idx3_hbm, emb_hbm, out_hbm, idxv, b0, b1, b2, b3,
                 g0, g1, g2, g3, s0, s1, s2, s3):
    bufs = (b0, b1, b2, b3)
    gsems = (g0, g1, g2, g3)
    ssems = (s0, s1, s2, s3)
    wid = lax.axis_index("s") * 2 + lax.axis_index("c")
    base = wid * GPW
    pltpu.sync_copy(idx3_hbm.at[wid], idxv)

    nj = GPW // 128
    gcp = [None] * nj
    scp = [None] * nj
    for j in range(nj):
        b = j % 4
        if j >= 4:
            scp[j - 4].wait()
        gcp[j] = pltpu.async_copy(emb_hbm.at[idxv.at[j]], bufs[b], gsems[b])
        if j >= 1:
            gcp[j - 1].wait()
            scp[j - 1] = pltpu.async_copy(
                bufs[(j - 1) % 4], out_hbm.at[pl.ds(base + (j - 1) * 128, 128)],
                ssems[(j - 1) % 4])
    gcp[nj - 1].wait()
    scp[nj - 1] = pltpu.async_copy(
        bufs[(nj - 1) % 4], out_hbm.at[pl.ds(base + (nj - 1) * 128, 128)],
        ssems[(nj - 1) % 4])
    for j in range(nj - 4, nj):
        scp[j].wait()


@functools.cache
def _get_stage3():
    return pl.kernel(
        _stage3_body,
        out_type=jax.ShapeDtypeStruct((NGATH, EMB), jnp.float32),
        mesh=plsc.VectorSubcoreMesh(core_axis_name="c", subcore_axis_name="s",
                                    num_cores=2, num_subcores=16),
        compiler_params=pltpu.CompilerParams(needs_layout_passes=False, use_tc_tiling_on_sc=False),
        scratch_types=(
            [pltpu.VMEM((GPW // 128, 128), jnp.int32)]
            + [pltpu.VMEM((128, EMB), jnp.float32)] * 4
            + [pltpu.SemaphoreType.DMA] * 8
        ),
    )


def _mm(x, w):
    return lax.dot_general(x, w, (((1,), (1,)), ((), ())),
                           preferred_element_type=jnp.float32)


def _pd(a, b):
    return jnp.sqrt(jnp.sum((a - b + EPS) ** 2, axis=1, keepdims=True))


def _stage4_body(*refs):
    (s0, s1, a0, a1, h0, h1, m0, m1) = refs[0:8]
    nsyn = refs[8:18]
    nhyp = refs[18:28]
    nmer = refs[28:38]
    (msk_s, msk_a, msk_h, msk_m) = refs[38:42]
    (wsyn, whyp, wmer, rh, bh, rm, bm) = refs[42:49]
    out_ref = refs[49]
    acc = refs[50]

    i = pl.program_id(0)

    @pl.when(i == 0)
    def _():
        for k in range(8):
            acc[k] = 0.0

    ws = wsyn[...]
    wh = whyp[...]
    wm = wmer[...]
    mh = lax.dot_general(rh[...], wh, (((1,), (0,)), ((), ())),
                         preferred_element_type=jnp.float32)
    mm_ = lax.dot_general(rm[...], wm, (((1,), (0,)), ((), ())),
                          preferred_element_type=jnp.float32)
    bhv = bh[...]
    bmv = bm[...]

    fm_s = 1.0 - (msk_s[...] == 0).astype(jnp.float32)
    fm_a = 1.0 - (msk_a[...] == 0).astype(jnp.float32)
    fm_h = 1.0 - (msk_h[...] == 0).astype(jnp.float32)
    fm_m = 1.0 - (msk_m[...] == 0).astype(jnp.float32)

    e1 = _mm(s0[...], ws)
    e2 = _mm(s1[...], ws)
    an = jnp.zeros_like(e1[:, :1])
    af = jnp.zeros_like(an)
    for j in range(NNEG):
        en = _mm(nsyn[j][...], ws)
        dn = _pd(e1, en)
        an = an + jnp.maximum(0.1 - dn, 0.0)
        af = af + jnp.maximum(dn - 1.5, 0.0)
    t_syn = (_pd(e1, e2) + an / NNEG + af / NNEG) * fm_s
    acc[0] = acc[0] + jnp.sum(t_syn)
    acc[1] = acc[1] + jnp.sum(fm_s)

    aa1 = _mm(a0[...], ws)
    aa2 = _mm(a1[...], ws)
    t_ant = jnp.maximum(1.0 - _pd(aa1, aa2), 0.0) * fm_a
    acc[2] = acc[2] + jnp.sum(t_ant)
    acc[3] = acc[3] + jnp.sum(fm_a)

    hh1 = _mm(h0[...], mh) + bhv
    hh2 = _mm(h1[...], wh)
    hn_acc = jnp.zeros_like(an)
    for j in range(NNEG):
        hn = _mm(nhyp[j][...], mh) + bhv
        hdn = _pd(hh2, hn)
        hn_acc = hn_acc + jnp.maximum(0.1 - hdn, 0.0)
    t_hyp = (_pd(hh1, hh2) + 3.0 * (hn_acc / NNEG)) * fm_h
    acc[4] = acc[4] + jnp.sum(t_hyp)
    acc[5] = acc[5] + jnp.sum(fm_h)

    mm1 = _mm(m0[...], mm_) + bmv
    mm2 = _mm(m1[...], wm)
    mn_acc = jnp.zeros_like(an)
    for j in range(NNEG):
        mn = _mm(nmer[j][...], mm_) + bmv
        mdn = _pd(mm2, mn)
        mn_acc = mn_acc + jnp.maximum(0.1 - mdn, 0.0)
    t_mer = (_pd(mm1, mm2) + mn_acc / NNEG) * fm_m
    acc[6] = acc[6] + jnp.sum(t_mer)
    acc[7] = acc[7] + jnp.sum(fm_m)

    @pl.when(i == pl.num_programs(0) - 1)
    def _():
        l0 = acc[0] / jnp.maximum(acc[1], 1.0)
        l1 = acc[2] / jnp.maximum(acc[3], 1.0)
        l2 = acc[4] / jnp.maximum(acc[5], 1.0)
        l3 = acc[6] / jnp.maximum(acc[7], 1.0)
        lane = lax.broadcasted_iota(jnp.int32, (1, 128), 1)
        outv = (jnp.where(lane == 0, l0, 0.0) + jnp.where(lane == 1, l1, 0.0)
                + jnp.where(lane == 2, l2, 0.0) + jnp.where(lane == 3, l3, 0.0))
        out_ref[...] = outv.astype(jnp.float32)


def _build_stage4(interpret=False):
    bs = 512
    grid = (B // bs,)
    gspec = lambda c: pl.BlockSpec((bs, EMB), lambda i, c=c: (c * (B // bs) + i, 0))
    in_specs = []
    for c in range(8):
        in_specs.append(gspec(c))
    for r in range(3):
        for j in range(NNEG):
            off = (NPAIR + r * B * NNEG + j * B) // bs
            in_specs.append(pl.BlockSpec((bs, EMB), lambda i, off=off: (off + i, 0)))
    for _ in range(4):
        in_specs.append(pl.BlockSpec((bs, 1), lambda i: (i, 0)))
    for _ in range(3):
        in_specs.append(pl.BlockSpec((WN, EMB), lambda i: (0, 0)))
    in_specs.append(pl.BlockSpec((WN, WN), lambda i: (0, 0)))
    in_specs.append(pl.BlockSpec((1, WN), lambda i: (0, 0)))
    in_specs.append(pl.BlockSpec((WN, WN), lambda i: (0, 0)))
    in_specs.append(pl.BlockSpec((1, WN), lambda i: (0, 0)))
    return pl.pallas_call(
        _stage4_body,
        grid=grid,
        in_specs=in_specs,
        out_specs=pl.BlockSpec((1, 128), lambda i: (0, 0)),
        out_shape=jax.ShapeDtypeStruct((1, 128), jnp.float32),
        scratch_shapes=[pltpu.SMEM((8,), jnp.float32)],
        compiler_params=pltpu.CompilerParams(vmem_limit_bytes=100 * 1024 * 1024),
        interpret=interpret,
    )


_stage4 = _build_stage4()


def kernel(synonyms, antonyms, hypernyms, meronyms, emb_table, vocab_freq,
           syn_proj_w, hypn_proj_w, mern_proj_w,
           hypn_rel_w, hypn_rel_b, mern_rel_w, mern_rel_b):
    skey = jax.random.key(42)
    us = [jax.random.uniform(jax.random.fold_in(skey, i), (B, NNEG), jnp.float32)
          for i in (1, 2, 3)]
    u_all = jnp.concatenate([u.T.reshape(-1) for u in us]).reshape(3840, 128)

    f_pad = jnp.concatenate(
        [vocab_freq, jnp.zeros((VP - V,), jnp.float32)]).reshape(64, 128, 128)

    p3, c3, r2 = _stage1(f_pad, u_all)
    p2d = p3.reshape(NROW32, 32)
    c1 = c3.reshape(NROW32)
    r1 = r2.reshape(NQ)

    nidx = _get_stage2()(p2d, c1, r1)

    pairs = jnp.stack([synonyms, antonyms, hypernyms, meronyms])
    pairs = pairs.transpose(0, 2, 1).reshape(-1).astype(jnp.int32)
    idx3 = jnp.concatenate([pairs, nidx]).reshape(NW, GPW // 128, 128)

    g = _get_stage3()(idx3, emb_table)

    gargs = [g] * 38
    margs = [synonyms[:, :1], antonyms[:, :1], hypernyms[:, :1], meronyms[:, :1]]
    wargs = [syn_proj_w, hypn_proj_w, mern_proj_w,
             hypn_rel_w, hypn_rel_b.reshape(1, WN),
             mern_rel_w, mern_rel_b.reshape(1, WN)]
    out2d = _stage4(*gargs, *margs, *wargs)
    return out2d[0, :4]


# speedup vs baseline: 8.3677x; 1.0088x over previous
import functools

import jax
import jax.numpy as jnp
from jax import lax
from jax.experimental import pallas as pl
from jax.experimental.pallas import tpu as pltpu
from jax.experimental.pallas import tpu_sc as plsc

V = 1000000
VP = 1 << 20
EMB = 64
WN = 64
B = 16384
NNEG = 10
NQ = 3 * B * NNEG
EPS = 1e-6

NROW32 = VP // 32
NW = 32
QPW = NQ // NW
CQ = 1920
NCH = QPW // CQ
NPAIR = 8 * B
NGATH = NPAIR + NQ
GPW = NGATH // NW


def _stage1_body(f_ref, u_ref, p_ref, c_ref, r_ref, carry_ref):
    i = pl.program_id(0)

    @pl.when(i == 0)
    def _():
        carry_ref[0] = 0.0

    x = f_ref[0]
    g = jnp.where(x > 0.0, jnp.exp(0.75 * jnp.log(jnp.where(x > 0.0, x, 1.0))), 0.0)

    ri = lax.broadcasted_iota(jnp.int32, (128, 128), 0)
    ci = lax.broadcasted_iota(jnp.int32, (128, 128), 1)
    upper = (ri <= ci).astype(jnp.float32)
    lstrict = (ri > ci).astype(jnp.float32)

    pref = lax.dot_general(g, upper, (((1,), (0,)), ((), ())),
                           precision=lax.Precision.HIGHEST,
                           preferred_element_type=jnp.float32)
    rowsum = jnp.sum(g, axis=1, keepdims=True)
    roff = lax.dot_general(lstrict, rowsum, (((1,), (0,)), ((), ())),
                           precision=lax.Precision.HIGHEST,
                           preferred_element_type=jnp.float32)
    carry = carry_ref[0]
    p = pref + roff + carry
    p_ref[0] = p

    r8 = lax.broadcasted_iota(jnp.int32, (128, 4), 0)
    c8 = lax.broadcasted_iota(jnp.int32, (128, 4), 1)
    esel = (r8 == 32 * c8 + 31).astype(jnp.float32)
    c_ref[0] = lax.dot_general(p, esel, (((1,), (0,)), ((), ())),
                               precision=lax.Precision.HIGHEST,
                               preferred_element_type=jnp.float32)

    carry_ref[0] = carry + jnp.sum(g)

    @pl.when(i == 63)
    def _():
        t = carry_ref[0]
        r_ref[...] = t * (1.0 - u_ref[...])


def _build_stage1(interpret=False):
    return pl.pallas_call(
        _stage1_body,
        grid=(64,),
        in_specs=[
            pl.BlockSpec((1, 128, 128), lambda i: (i, 0, 0)),
            pl.BlockSpec((3840, 128), lambda i: (0, 0)),
        ],
        out_specs=[
            pl.BlockSpec((1, 128, 128), lambda i: (i, 0, 0)),
            pl.BlockSpec((1, 128, 4), lambda i: (i, 0, 0)),
            pl.BlockSpec((3840, 128), lambda i: (0, 0)),
        ],
        out_shape=[
            jax.ShapeDtypeStruct((64, 128, 128), jnp.float32),
            jax.ShapeDtypeStruct((64, 128, 4), jnp.float32),
            jax.ShapeDtypeStruct((3840, 128), jnp.float32),
        ],
        scratch_shapes=[pltpu.SMEM((1,), jnp.float32)],
        interpret=interpret,
    )


_stage1 = _build_stage1()


def _stage2_body(p2d_hbm, c_hbm, r_hbm, out_hbm, cv, qv, rowv2, rowsbuf, outv, sem):
    wid = lax.axis_index("s") * 2 + lax.axis_index("c")
    base = wid * QPW
    pltpu.sync_copy(c_hbm, cv)

    def chunk_body(ch, _):
        qoff = base + ch * CQ
        pltpu.sync_copy(r_hbm.at[pl.ds(qoff, CQ)], qv)

        def coarse_body(j2, _):
            for k in range(8):
                q = qv[pl.ds(j2 * 128 + k * 16, 16)]
                lo = jnp.zeros((16,), jnp.int32)
                for st in range(15):
                    h = 1 << (14 - st)
                    probe = lo + (h - 1)
                    val = plsc.load_gather(cv, [probe])
                    lo = jnp.where(val < q, lo + h, lo)
                rowv2[j2, pl.ds(k * 16, 16)] = lo
            return 0

        lax.fori_loop(0, CQ // 128, coarse_body, 0)

        cps = [
            pltpu.async_copy(p2d_hbm.at[rowv2.at[j]], rowsbuf.at[pl.ds(j * 128, 128)], sem)
            for j in range(CQ // 128)
        ]
        for cp in cps:
            cp.wait()

        def refine_body(j2, _):
            for k in range(8):
                off = j2 * 128 + k * 16
                q = qv[pl.ds(off, 16)]
                lo = rowv2[j2, pl.ds(k * 16, 16)]
                lanesel = lax.iota(jnp.int32, 16) + off
                pos = jnp.zeros((16,), jnp.int32)
                for st in range(5):
                    h = 1 << (4 - st)
                    probe = pos + (h - 1)
                    vals = plsc.load_gather(rowsbuf, [lanesel, probe])
                    pos = jnp.where(vals < q, pos + h, pos)
                idx = jnp.minimum(lo * 32 + pos, V - 1)
                outv[pl.ds(off, 16)] = idx
            return 0

        lax.fori_loop(0, CQ // 128, refine_body, 0)
        pltpu.sync_copy(outv, out_hbm.at[pl.ds(qoff, CQ)])
        return 0

    lax.fori_loop(0, NCH, chunk_body, 0)


@functools.cache
def _get_stage2():
    return pl.kernel(
        _stage2_body,
        out_type=jax.ShapeDtypeStruct((NQ,), jnp.int32),
        mesh=plsc.VectorSubcoreMesh(core_axis_name="c", subcore_axis_name="s",
                                    num_cores=2, num_subcores=16),
        compiler_params=pltpu.CompilerParams(needs_layout_passes=False, use_tc_tiling_on_sc=False),
        scratch_types=[
            pltpu.VMEM((NROW32,), jnp.float32),
            pltpu.VMEM((CQ,), jnp.float32),
            pltpu.VMEM((CQ // 128, 128), jnp.int32),
            pltpu.VMEM((CQ, 32), jnp.float32),
            pltpu.VMEM((CQ,), jnp.int32),
            pltpu.SemaphoreType.DMA,
        ],
    )


def _stage3_body(idx3_hbm, emb_hbm, out_hbm, idxv, b0, b1, b2, b3,
                 g0, g1, g2, g3, s0, s1, s2, s3):
    bufs = (b0, b1, b2, b3)
    gsems = (g0, g1, g2, g3)
    ssems = (s0, s1, s2, s3)
    wid = lax.axis_index("s") * 2 + lax.axis_index("c")
    base = wid * GPW
    pltpu.sync_copy(idx3_hbm.at[wid], idxv)

    nj = GPW // 128
    gcp = [None] * nj
    scp = [None] * nj
    for j in range(nj):
        b = j % 4
        if j >= 4:
            scp[j - 4].wait()
        gcp[j] = pltpu.async_copy(emb_hbm.at[idxv.at[j]], bufs[b], gsems[b])
        if j >= 1:
            gcp[j - 1].wait()
            scp[j - 1] = pltpu.async_copy(
                bufs[(j - 1) % 4], out_hbm.at[pl.ds(base + (j - 1) * 128, 128)],
                ssems[(j - 1) % 4])
    gcp[nj - 1].wait()
    scp[nj - 1] = pltpu.async_copy(
        bufs[(nj - 1) % 4], out_hbm.at[pl.ds(base + (nj - 1) * 128, 128)],
        ssems[(nj - 1) % 4])
    for j in range(nj - 4, nj):
        scp[j].wait()


@functools.cache
def _get_stage3():
    return pl.kernel(
        _stage3_body,
        out_type=jax.ShapeDtypeStruct((NGATH, EMB), jnp.float32),
        mesh=plsc.VectorSubcoreMesh(core_axis_name="c", subcore_axis_name="s",
                                    num_cores=2, num_subcores=16),
        compiler_params=pltpu.CompilerParams(needs_layout_passes=False,
                                             use_tc_tiling_on_sc=False),
        scratch_types=(
            [pltpu.VMEM((GPW // 128, 128), jnp.int32)]
            + [pltpu.VMEM((128, EMB), jnp.float32)] * 4
            + [pltpu.SemaphoreType.DMA] * 8
        ),
    )


def _mm(x, w):
    return lax.dot_general(x, w, (((1,), (1,)), ((), ())),
                           preferred_element_type=jnp.float32)


def _pd(a, b):
    return jnp.sqrt(jnp.sum((a - b + EPS) ** 2, axis=1, keepdims=True))


def _stage4_body(*refs):
    (s0, s1, a0, a1, h0, h1, m0, m1) = refs[0:8]
    nsyn = refs[8:18]
    nhyp = refs[18:28]
    nmer = refs[28:38]
    (msk_s, msk_a, msk_h, msk_m) = refs[38:42]
    (wsyn, whyp, wmer, rh, bh, rm, bm) = refs[42:49]
    out_ref = refs[49]
    acc = refs[50]

    i = pl.program_id(0)

    @pl.when(i == 0)
    def _():
        for k in range(8):
            acc[k] = 0.0

    ws = wsyn[...]
    wh = whyp[...]
    wm = wmer[...]
    mh = lax.dot_general(rh[...], wh, (((1,), (0,)), ((), ())),
                         preferred_element_type=jnp.float32)
    mm_ = lax.dot_general(rm[...], wm, (((1,), (0,)), ((), ())),
                          preferred_element_type=jnp.float32)
    bhv = bh[...]
    bmv = bm[...]

    fm_s = 1.0 - (msk_s[...] == 0).astype(jnp.float32)
    fm_a = 1.0 - (msk_a[...] == 0).astype(jnp.float32)
    fm_h = 1.0 - (msk_h[...] == 0).astype(jnp.float32)
    fm_m = 1.0 - (msk_m[...] == 0).astype(jnp.float32)

    e1 = _mm(s0[...], ws)
    e2 = _mm(s1[...], ws)
    an = jnp.zeros_like(e1[:, :1])
    af = jnp.zeros_like(an)
    for j in range(NNEG):
        en = _mm(nsyn[j][...], ws)
        dn = _pd(e1, en)
        an = an + jnp.maximum(0.1 - dn, 0.0)
        af = af + jnp.maximum(dn - 1.5, 0.0)
    t_syn = (_pd(e1, e2) + an / NNEG + af / NNEG) * fm_s
    acc[0] = acc[0] + jnp.sum(t_syn)
    acc[1] = acc[1] + jnp.sum(fm_s)

    aa1 = _mm(a0[...], ws)
    aa2 = _mm(a1[...], ws)
    t_ant = jnp.maximum(1.0 - _pd(aa1, aa2), 0.0) * fm_a
    acc[2] = acc[2] + jnp.sum(t_ant)
    acc[3] = acc[3] + jnp.sum(fm_a)

    hh1 = _mm(h0[...], mh) + bhv
    hh2 = _mm(h1[...], wh)
    hn_acc = jnp.zeros_like(an)
    for j in range(NNEG):
        hn = _mm(nhyp[j][...], mh) + bhv
        hdn = _pd(hh2, hn)
        hn_acc = hn_acc + jnp.maximum(0.1 - hdn, 0.0)
    t_hyp = (_pd(hh1, hh2) + 3.0 * (hn_acc / NNEG)) * fm_h
    acc[4] = acc[4] + jnp.sum(t_hyp)
    acc[5] = acc[5] + jnp.sum(fm_h)

    mm1 = _mm(m0[...], mm_) + bmv
    mm2 = _mm(m1[...], wm)
    mn_acc = jnp.zeros_like(an)
    for j in range(NNEG):
        mn = _mm(nmer[j][...], mm_) + bmv
        mdn = _pd(mm2, mn)
        mn_acc = mn_acc + jnp.maximum(0.1 - mdn, 0.0)
    t_mer = (_pd(mm1, mm2) + mn_acc / NNEG) * fm_m
    acc[6] = acc[6] + jnp.sum(t_mer)
    acc[7] = acc[7] + jnp.sum(fm_m)

    @pl.when(i == pl.num_programs(0) - 1)
    def _():
        l0 = acc[0] / jnp.maximum(acc[1], 1.0)
        l1 = acc[2] / jnp.maximum(acc[3], 1.0)
        l2 = acc[4] / jnp.maximum(acc[5], 1.0)
        l3 = acc[6] / jnp.maximum(acc[7], 1.0)
        lane = lax.broadcasted_iota(jnp.int32, (1, 128), 1)
        outv = (jnp.where(lane == 0, l0, 0.0) + jnp.where(lane == 1, l1, 0.0)
                + jnp.where(lane == 2, l2, 0.0) + jnp.where(lane == 3, l3, 0.0))
        out_ref[...] = outv.astype(jnp.float32)


def _build_stage4(interpret=False):
    bs = 1024
    grid = (B // bs,)
    gspec = lambda c: pl.BlockSpec((bs, EMB), lambda i, c=c: (c * (B // bs) + i, 0))
    in_specs = []
    for c in range(8):
        in_specs.append(gspec(c))
    for r in range(3):
        for j in range(NNEG):
            off = (NPAIR + r * B * NNEG + j * B) // bs
            in_specs.append(pl.BlockSpec((bs, EMB), lambda i, off=off: (off + i, 0)))
    for _ in range(4):
        in_specs.append(pl.BlockSpec((bs, 1), lambda i: (i, 0)))
    for _ in range(3):
        in_specs.append(pl.BlockSpec((WN, EMB), lambda i: (0, 0)))
    in_specs.append(pl.BlockSpec((WN, WN), lambda i: (0, 0)))
    in_specs.append(pl.BlockSpec((1, WN), lambda i: (0, 0)))
    in_specs.append(pl.BlockSpec((WN, WN), lambda i: (0, 0)))
    in_specs.append(pl.BlockSpec((1, WN), lambda i: (0, 0)))
    return pl.pallas_call(
        _stage4_body,
        grid=grid,
        in_specs=in_specs,
        out_specs=pl.BlockSpec((1, 128), lambda i: (0, 0)),
        out_shape=jax.ShapeDtypeStruct((1, 128), jnp.float32),
        scratch_shapes=[pltpu.SMEM((8,), jnp.float32)],
        compiler_params=pltpu.CompilerParams(vmem_limit_bytes=100 * 1024 * 1024),
        interpret=interpret,
    )


_stage4 = _build_stage4()


def kernel(synonyms, antonyms, hypernyms, meronyms, emb_table, vocab_freq,
           syn_proj_w, hypn_proj_w, mern_proj_w,
           hypn_rel_w, hypn_rel_b, mern_rel_w, mern_rel_b):
    skey = jax.random.key(42)
    us = [jax.random.uniform(jax.random.fold_in(skey, i), (B, NNEG), jnp.float32)
          for i in (1, 2, 3)]
    u_all = jnp.concatenate([u.T.reshape(-1) for u in us]).reshape(3840, 128)

    f_pad = jnp.concatenate(
        [vocab_freq, jnp.zeros((VP - V,), jnp.float32)]).reshape(64, 128, 128)

    p3, c3, r2 = _stage1(f_pad, u_all)
    p2d = p3.reshape(NROW32, 32)
    c1 = c3.reshape(NROW32)
    r1 = r2.reshape(NQ)

    nidx = _get_stage2()(p2d, c1, r1)

    pairs = jnp.stack([synonyms, antonyms, hypernyms, meronyms])
    pairs = pairs.transpose(0, 2, 1).reshape(-1).astype(jnp.int32)
    idx3 = jnp.concatenate([pairs, nidx]).reshape(NW, GPW // 128, 128)

    g = _get_stage3()(idx3, emb_table)

    gargs = [g] * 38
    margs = [synonyms[:, :1], antonyms[:, :1], hypernyms[:, :1], meronyms[:, :1]]
    wargs = [syn_proj_w, hypn_proj_w, mern_proj_w,
             hypn_rel_w, hypn_rel_b.reshape(1, WN),
             mern_rel_w, mern_rel_b.reshape(1, WN)]
    out2d = _stage4(*gargs, *margs, *wargs)
    return out2d[0, :4]


# speedup vs baseline: 8.3727x; 1.0006x over previous
import functools

import jax
import jax.numpy as jnp
from jax import lax
from jax.experimental import pallas as pl
from jax.experimental.pallas import tpu as pltpu
from jax.experimental.pallas import tpu_sc as plsc

V = 1000000
VP = 1 << 20
EMB = 64
WN = 64
B = 16384
NNEG = 10
NQ = 3 * B * NNEG
NROW = VP // 16
EPS = 1e-6

NROW32 = VP // 32
NW = 32
QPW = NQ // NW
CQ = 1536
NCH = QPW // CQ
NPAIR = 8 * B
NGATH = NPAIR + NQ
GPW = NGATH // NW


def _stage1_body(f_ref, u_ref, p_ref, c_ref, r_ref, carry_ref):
    i = pl.program_id(0)

    @pl.when(i == 0)
    def _():
        carry_ref[0] = 0.0

    x = f_ref[0]
    g = jnp.where(x > 0.0, jnp.exp(0.75 * jnp.log(jnp.where(x > 0.0, x, 1.0))), 0.0)

    ri = lax.broadcasted_iota(jnp.int32, (128, 128), 0)
    ci = lax.broadcasted_iota(jnp.int32, (128, 128), 1)
    upper = (ri <= ci).astype(jnp.float32)
    lstrict = (ri > ci).astype(jnp.float32)

    pref = lax.dot_general(g, upper, (((1,), (0,)), ((), ())),
                           precision=lax.Precision.HIGHEST,
                           preferred_element_type=jnp.float32)
    rowsum = jnp.sum(g, axis=1, keepdims=True)
    roff = lax.dot_general(lstrict, rowsum, (((1,), (0,)), ((), ())),
                           precision=lax.Precision.HIGHEST,
                           preferred_element_type=jnp.float32)
    carry = carry_ref[0]
    p = pref + roff + carry
    p_ref[0] = p

    r8 = lax.broadcasted_iota(jnp.int32, (128, 4), 0)
    c8 = lax.broadcasted_iota(jnp.int32, (128, 4), 1)
    esel = (r8 == 32 * c8 + 31).astype(jnp.float32)
    c_ref[0] = lax.dot_general(p, esel, (((1,), (0,)), ((), ())),
                               precision=lax.Precision.HIGHEST,
                               preferred_element_type=jnp.float32)

    carry_ref[0] = carry + jnp.sum(g)

    @pl.when(i == 63)
    def _():
        t = carry_ref[0]
        r_ref[...] = t * (1.0 - u_ref[...])


def _build_stage1(interpret=False):
    return pl.pallas_call(
        _stage1_body,
        grid=(64,),
        in_specs=[
            pl.BlockSpec((1, 128, 128), lambda i: (i, 0, 0)),
            pl.BlockSpec((3840, 128), lambda i: (0, 0)),
        ],
        out_specs=[
            pl.BlockSpec((1, 128, 128), lambda i: (i, 0, 0)),
            pl.BlockSpec((1, 128, 4), lambda i: (i, 0, 0)),
            pl.BlockSpec((3840, 128), lambda i: (0, 0)),
        ],
        out_shape=[
            jax.ShapeDtypeStruct((64, 128, 128), jnp.float32),
            jax.ShapeDtypeStruct((64, 128, 4), jnp.float32),
            jax.ShapeDtypeStruct((3840, 128), jnp.float32),
        ],
        scratch_shapes=[pltpu.SMEM((1,), jnp.float32)],
        interpret=interpret,
    )


_stage1 = _build_stage1()


def _stage2_body(p2d_hbm, c_hbm, r_hbm, out_hbm, cv, qv, rowv2, rowsbuf, outv, sem):
    wid = lax.axis_index("s") * 2 + lax.axis_index("c")
    base = wid * QPW
    pltpu.sync_copy(c_hbm, cv)

    def chunk_body(ch, _):
        qoff = base + ch * CQ
        pltpu.sync_copy(r_hbm.at[pl.ds(qoff, CQ)], qv)

        def coarse_body(j2, _):
            qs = [qv[pl.ds(j2 * 128 + k * 16, 16)] for k in range(8)]
            los = [jnp.zeros((16,), jnp.int32) for _ in range(8)]
            for st in range(15):
                h = 1 << (14 - st)
                vals = [plsc.load_gather(cv, [los[k] + (h - 1)]) for k in range(8)]
                for k in range(8):
                    los[k] = jnp.where(vals[k] < qs[k], los[k] + h, los[k])
            for k in range(8):
                rowv2[j2, pl.ds(k * 16, 16)] = los[k]
            return 0

        lax.fori_loop(0, 12, coarse_body, 0)

        cps = [
            pltpu.async_copy(p2d_hbm.at[rowv2.at[j]], rowsbuf.at[pl.ds(j * 128, 128)], sem)
            for j in range(12)
        ]
        for cp in cps:
            cp.wait()

        def refine_body(j2, _):
            qs = [qv[pl.ds(j2 * 128 + k * 16, 16)] for k in range(8)]
            lsel = [lax.iota(jnp.int32, 16) + (j2 * 128 + k * 16) for k in range(8)]
            poss = [jnp.zeros((16,), jnp.int32) for _ in range(8)]
            for st in range(5):
                h = 1 << (4 - st)
                vals = [plsc.load_gather(rowsbuf, [lsel[k], poss[k] + (h - 1)])
                        for k in range(8)]
                for k in range(8):
                    poss[k] = jnp.where(vals[k] < qs[k], poss[k] + h, poss[k])
            for k in range(8):
                lo = rowv2[j2, pl.ds(k * 16, 16)]
                outv[pl.ds(j2 * 128 + k * 16, 16)] = jnp.minimum(
                    lo * 32 + poss[k], V - 1)
            return 0

        lax.fori_loop(0, 12, refine_body, 0)
        pltpu.sync_copy(outv, out_hbm.at[pl.ds(qoff, CQ)])
        return 0

    lax.fori_loop(0, NCH, chunk_body, 0)


@functools.cache
def _get_stage2():
    return pl.kernel(
        _stage2_body,
        out_type=jax.ShapeDtypeStruct((NQ,), jnp.int32),
        mesh=plsc.VectorSubcoreMesh(core_axis_name="c", subcore_axis_name="s",
                                    num_cores=2, num_subcores=16),
        compiler_params=pltpu.CompilerParams(needs_layout_passes=False, use_tc_tiling_on_sc=False),
        scratch_types=[
            pltpu.VMEM((NROW32,), jnp.float32),
            pltpu.VMEM((CQ,), jnp.float32),
            pltpu.VMEM((12, 128), jnp.int32),
            pltpu.VMEM((CQ, 32), jnp.float32),
            pltpu.VMEM((CQ,), jnp.int32),
            pltpu.SemaphoreType.DMA,
        ],
    )


def _stage3_body(idx3_hbm, emb_hbm, out_hbm, idxv, b0, b1, b2, b3,
                 g0, g1, g2, g3, s0, s1, s2, s3):
    bufs = (b0, b1, b2, b3)
    gsems = (g0, g1, g2, g3)
    ssems = (s0, s1, s2, s3)
    wid = lax.axis_index("s") * 2 + lax.axis_index("c")
    base = wid * GPW
    pltpu.sync_copy(idx3_hbm.at[wid], idxv)

    nj = GPW // 128
    gcp = [None] * nj
    scp = [None] * nj
    for j in range(nj):
        b = j % 4
        if j >= 4:
            scp[j - 4].wait()
        gcp[j] = pltpu.async_copy(emb_hbm.at[idxv.at[j]], bufs[b], gsems[b])
        if j >= 1:
            gcp[j - 1].wait()
            scp[j - 1] = pltpu.async_copy(
                bufs[(j - 1) % 4], out_hbm.at[pl.ds(base + (j - 1) * 128, 128)],
                ssems[(j - 1) % 4])
    gcp[nj - 1].wait()
    scp[nj - 1] = pltpu.async_copy(
        bufs[(nj - 1) % 4], out_hbm.at[pl.ds(base + (nj - 1) * 128, 128)],
        ssems[(nj - 1) % 4])
    for j in range(nj - 4, nj):
        scp[j].wait()


@functools.cache
def _get_stage3():
    return pl.kernel(
        _stage3_body,
        out_type=jax.ShapeDtypeStruct((NGATH, EMB), jnp.float32),
        mesh=plsc.VectorSubcoreMesh(core_axis_name="c", subcore_axis_name="s",
                                    num_cores=2, num_subcores=16),
        compiler_params=pltpu.CompilerParams(needs_layout_passes=False,
                                             use_tc_tiling_on_sc=False),
        scratch_types=(
            [pltpu.VMEM((GPW // 128, 128), jnp.int32)]
            + [pltpu.VMEM((128, EMB), jnp.float32)] * 4
            + [pltpu.SemaphoreType.DMA] * 8
        ),
    )


def _mm(x, w):
    return lax.dot_general(x, w, (((1,), (1,)), ((), ())),
                           preferred_element_type=jnp.float32)


def _pd(a, b):
    return jnp.sqrt(jnp.sum((a - b + EPS) ** 2, axis=1, keepdims=True))


def _stage4_body(*refs):
    (s0, s1, a0, a1, h0, h1, m0, m1) = refs[0:8]
    nsyn = refs[8:18]
    nhyp = refs[18:28]
    nmer = refs[28:38]
    (msk_s, msk_a, msk_h, msk_m) = refs[38:42]
    (wsyn, whyp, wmer, rh, bh, rm, bm) = refs[42:49]
    out_ref = refs[49]
    acc = refs[50]

    i = pl.program_id(0)

    @pl.when(i == 0)
    def _():
        for k in range(8):
            acc[k] = 0.0

    ws = wsyn[...]
    wh = whyp[...]
    wm = wmer[...]
    mh = lax.dot_general(rh[...], wh, (((1,), (0,)), ((), ())),
                         preferred_element_type=jnp.float32)
    mm_ = lax.dot_general(rm[...], wm, (((1,), (0,)), ((), ())),
                          preferred_element_type=jnp.float32)
    bhv = bh[...]
    bmv = bm[...]

    fm_s = 1.0 - (msk_s[...] == 0).astype(jnp.float32)
    fm_a = 1.0 - (msk_a[...] == 0).astype(jnp.float32)
    fm_h = 1.0 - (msk_h[...] == 0).astype(jnp.float32)
    fm_m = 1.0 - (msk_m[...] == 0).astype(jnp.float32)

    e1 = _mm(s0[...], ws)
    e2 = _mm(s1[...], ws)
    an = jnp.zeros_like(e1[:, :1])
    af = jnp.zeros_like(an)
    for j in range(NNEG):
        en = _mm(nsyn[j][...], ws)
        dn = _pd(e1, en)
        an = an + jnp.maximum(0.1 - dn, 0.0)
        af = af + jnp.maximum(dn - 1.5, 0.0)
    t_syn = (_pd(e1, e2) + an / NNEG + af / NNEG) * fm_s
    acc[0] = acc[0] + jnp.sum(t_syn)
    acc[1] = acc[1] + jnp.sum(fm_s)

    aa1 = _mm(a0[...], ws)
    aa2 = _mm(a1[...], ws)
    t_ant = jnp.maximum(1.0 - _pd(aa1, aa2), 0.0) * fm_a
    acc[2] = acc[2] + jnp.sum(t_ant)
    acc[3] = acc[3] + jnp.sum(fm_a)

    hh1 = _mm(h0[...], mh) + bhv
    hh2 = _mm(h1[...], wh)
    hn_acc = jnp.zeros_like(an)
    for j in range(NNEG):
        hn = _mm(nhyp[j][...], mh) + bhv
        hdn = _pd(hh2, hn)
        hn_acc = hn_acc + jnp.maximum(0.1 - hdn, 0.0)
    t_hyp = (_pd(hh1, hh2) + 3.0 * (hn_acc / NNEG)) * fm_h
    acc[4] = acc[4] + jnp.sum(t_hyp)
    acc[5] = acc[5] + jnp.sum(fm_h)

    mm1 = _mm(m0[...], mm_) + bmv
    mm2 = _mm(m1[...], wm)
    mn_acc = jnp.zeros_like(an)
    for j in range(NNEG):
        mn = _mm(nmer[j][...], mm_) + bmv
        mdn = _pd(mm2, mn)
        mn_acc = mn_acc + jnp.maximum(0.1 - mdn, 0.0)
    t_mer = (_pd(mm1, mm2) + mn_acc / NNEG) * fm_m
    acc[6] = acc[6] + jnp.sum(t_mer)
    acc[7] = acc[7] + jnp.sum(fm_m)

    @pl.when(i == pl.num_programs(0) - 1)
    def _():
        l0 = acc[0] / jnp.maximum(acc[1], 1.0)
        l1 = acc[2] / jnp.maximum(acc[3], 1.0)
        l2 = acc[4] / jnp.maximum(acc[5], 1.0)
        l3 = acc[6] / jnp.maximum(acc[7], 1.0)
        lane = lax.broadcasted_iota(jnp.int32, (1, 128), 1)
        outv = (jnp.where(lane == 0, l0, 0.0) + jnp.where(lane == 1, l1, 0.0)
                + jnp.where(lane == 2, l2, 0.0) + jnp.where(lane == 3, l3, 0.0))
        out_ref[...] = outv.astype(jnp.float32)


def _build_stage4(interpret=False):
    bs = 1024
    grid = (B // bs,)
    gspec = lambda c: pl.BlockSpec((bs, EMB), lambda i, c=c: (c * (B // bs) + i, 0))
    in_specs = []
    for c in range(8):
        in_specs.append(gspec(c))
    for r in range(3):
        for j in range(NNEG):
            off = (NPAIR + r * B * NNEG + j * B) // bs
            in_specs.append(pl.BlockSpec((bs, EMB), lambda i, off=off: (off + i, 0)))
    for _ in range(4):
        in_specs.append(pl.BlockSpec((bs, 1), lambda i: (i, 0)))
    for _ in range(3):
        in_specs.append(pl.BlockSpec((WN, EMB), lambda i: (0, 0)))
    in_specs.append(pl.BlockSpec((WN, WN), lambda i: (0, 0)))
    in_specs.append(pl.BlockSpec((1, WN), lambda i: (0, 0)))
    in_specs.append(pl.BlockSpec((WN, WN), lambda i: (0, 0)))
    in_specs.append(pl.BlockSpec((1, WN), lambda i: (0, 0)))
    return pl.pallas_call(
        _stage4_body,
        grid=grid,
        in_specs=in_specs,
        out_specs=pl.BlockSpec((1, 128), lambda i: (0, 0)),
        out_shape=jax.ShapeDtypeStruct((1, 128), jnp.float32),
        scratch_shapes=[pltpu.SMEM((8,), jnp.float32)],
        compiler_params=pltpu.CompilerParams(vmem_limit_bytes=100 * 1024 * 1024),
        interpret=interpret,
    )


_stage4 = _build_stage4()


def kernel(synonyms, antonyms, hypernyms, meronyms, emb_table, vocab_freq,
           syn_proj_w, hypn_proj_w, mern_proj_w,
           hypn_rel_w, hypn_rel_b, mern_rel_w, mern_rel_b):
    skey = jax.random.key(42)
    us = [jax.random.uniform(jax.random.fold_in(skey, i), (B, NNEG), jnp.float32)
          for i in (1, 2, 3)]
    u_all = jnp.concatenate([u.T.reshape(-1) for u in us]).reshape(3840, 128)

    f_pad = jnp.concatenate(
        [vocab_freq, jnp.zeros((VP - V,), jnp.float32)]).reshape(64, 128, 128)

    p3, c3, r2 = _stage1(f_pad, u_all)
    p2d = p3.reshape(NROW32, 32)
    c1 = c3.reshape(NROW32)
    r1 = r2.reshape(NQ)

    nidx = _get_stage2()(p2d, c1, r1)

    pairs = jnp.stack([synonyms, antonyms, hypernyms, meronyms])
    pairs = pairs.transpose(0, 2, 1).reshape(-1).astype(jnp.int32)
    idx3 = jnp.concatenate([pairs, nidx]).reshape(NW, GPW // 128, 128)

    g = _get_stage3()(idx3, emb_table)

    gargs = [g] * 38
    margs = [synonyms[:, :1], antonyms[:, :1], hypernyms[:, :1], meronyms[:, :1]]
    wargs = [syn_proj_w, hypn_proj_w, mern_proj_w,
             hypn_rel_w, hypn_rel_b.reshape(1, WN),
             mern_rel_w, mern_rel_b.reshape(1, WN)]
    out2d = _stage4(*gargs, *margs, *wargs)
    return out2d[0, :4]


# speedup vs baseline: 8.3782x; 1.0007x over previous
import functools

import jax
import jax.numpy as jnp
from jax import lax
from jax.experimental import pallas as pl
from jax.experimental.pallas import tpu as pltpu
from jax.experimental.pallas import tpu_sc as plsc

V = 1000000
VP = 1 << 20
EMB = 64
WN = 64
B = 16384
NNEG = 10
NQ = 3 * B * NNEG
EPS = 1e-6

NROW32 = VP // 32
NW = 32
QPW = NQ // NW
CQ = 1536
NCH = QPW // CQ
NPAIR = 8 * B
NGATH = NPAIR + NQ
GPW = NGATH // NW


def _stage1_body(f_ref, u_ref, p_ref, c_ref, r_ref, carry_ref):
    i = pl.program_id(0)

    @pl.when(i == 0)
    def _():
        carry_ref[0] = 0.0

    x = f_ref[0]
    g = jnp.where(x > 0.0, jnp.exp(0.75 * jnp.log(jnp.where(x > 0.0, x, 1.0))), 0.0)

    ri = lax.broadcasted_iota(jnp.int32, (128, 128), 0)
    ci = lax.broadcasted_iota(jnp.int32, (128, 128), 1)
    upper = (ri <= ci).astype(jnp.float32)
    lstrict = (ri > ci).astype(jnp.float32)

    pref = lax.dot_general(g, upper, (((1,), (0,)), ((), ())),
                           precision=lax.Precision.HIGHEST,
                           preferred_element_type=jnp.float32)
    rowsum = jnp.sum(g, axis=1, keepdims=True)
    roff = lax.dot_general(lstrict, rowsum, (((1,), (0,)), ((), ())),
                           precision=lax.Precision.HIGHEST,
                           preferred_element_type=jnp.float32)
    carry = carry_ref[0]
    p = pref + roff + carry
    p_ref[0] = p

    r8 = lax.broadcasted_iota(jnp.int32, (128, 4), 0)
    c8 = lax.broadcasted_iota(jnp.int32, (128, 4), 1)
    esel = (r8 == 32 * c8 + 31).astype(jnp.float32)
    c_ref[0] = lax.dot_general(p, esel, (((1,), (0,)), ((), ())),
                               precision=lax.Precision.HIGHEST,
                               preferred_element_type=jnp.float32)

    carry_ref[0] = carry + jnp.sum(g)

    @pl.when(i == 63)
    def _():
        t = carry_ref[0]
        r_ref[...] = t * (1.0 - u_ref[...])


def _build_stage1(interpret=False):
    return pl.pallas_call(
        _stage1_body,
        grid=(64,),
        in_specs=[
            pl.BlockSpec((1, 128, 128), lambda i: (i, 0, 0)),
            pl.BlockSpec((3840, 128), lambda i: (0, 0)),
        ],
        out_specs=[
            pl.BlockSpec((1, 128, 128), lambda i: (i, 0, 0)),
            pl.BlockSpec((1, 128, 4), lambda i: (i, 0, 0)),
            pl.BlockSpec((3840, 128), lambda i: (0, 0)),
        ],
        out_shape=[
            jax.ShapeDtypeStruct((64, 128, 128), jnp.float32),
            jax.ShapeDtypeStruct((64, 128, 4), jnp.float32),
            jax.ShapeDtypeStruct((3840, 128), jnp.float32),
        ],
        scratch_shapes=[pltpu.SMEM((1,), jnp.float32)],
        interpret=interpret,
    )


_stage1 = _build_stage1()


def _stage2_body(p2d_hbm, c_hbm, r_hbm, out_hbm, cv, qv, rowv2, rowsbuf, outv, sem):
    wid = lax.axis_index("s") * 2 + lax.axis_index("c")
    base = wid * QPW
    pltpu.sync_copy(c_hbm, cv)

    def chunk_body(ch, _):
        qoff = base + ch * CQ
        pltpu.sync_copy(r_hbm.at[pl.ds(qoff, CQ)], qv)

        def coarse_body(j2, _):
            for k in range(8):
                q = qv[pl.ds(j2 * 128 + k * 16, 16)]
                lo = jnp.zeros((16,), jnp.int32)
                for st in range(15):
                    h = 1 << (14 - st)
                    probe = lo + (h - 1)
                    val = plsc.load_gather(cv, [probe])
                    lo = jnp.where(val < q, lo + h, lo)
                rowv2[j2, pl.ds(k * 16, 16)] = lo
            return 0

        lax.fori_loop(0, 12, coarse_body, 0)

        cps = [
            pltpu.async_copy(p2d_hbm.at[rowv2.at[j]], rowsbuf.at[pl.ds(j * 128, 128)], sem)
            for j in range(12)
        ]
        for cp in cps:
            cp.wait()

        def refine_body(j2, _):
            for k in range(8):
                off = j2 * 128 + k * 16
                q = qv[pl.ds(off, 16)]
                lo = rowv2[j2, pl.ds(k * 16, 16)]
                lanesel = lax.iota(jnp.int32, 16) + off
                pos = jnp.zeros((16,), jnp.int32)
                for st in range(5):
                    h = 1 << (4 - st)
                    probe = pos + (h - 1)
                    vals = plsc.load_gather(rowsbuf, [lanesel, probe])
                    pos = jnp.where(vals < q, pos + h, pos)
                idx = jnp.minimum(lo * 32 + pos, V - 1)
                outv[pl.ds(off, 16)] = idx
            return 0

        lax.fori_loop(0, 12, refine_body, 0)
        pltpu.sync_copy(outv, out_hbm.at[pl.ds(qoff, CQ)])
        return 0

    lax.fori_loop(0, NCH, chunk_body, 0)


@functools.cache
def _get_stage2():
    return pl.kernel(
        _stage2_body,
        out_type=jax.ShapeDtypeStruct((NQ,), jnp.int32),
        mesh=plsc.VectorSubcoreMesh(core_axis_name="c", subcore_axis_name="s",
                                    num_cores=2, num_subcores=16),
        compiler_params=pltpu.CompilerParams(needs_layout_passes=False, use_tc_tiling_on_sc=False),
        scratch_types=[
            pltpu.VMEM((NROW32,), jnp.float32),
            pltpu.VMEM((CQ,), jnp.float32),
            pltpu.VMEM((12, 128), jnp.int32),
            pltpu.VMEM((CQ, 32), jnp.float32),
            pltpu.VMEM((CQ,), jnp.int32),
            pltpu.SemaphoreType.DMA,
        ],
    )


def _stage3_body(idx3_hbm, emb_hbm, out_hbm, idxv, b0, b1, b2, b3,
                 g0, g1, g2, g3, s0, s1, s2, s3):
    bufs = (b0, b1, b2, b3)
    gsems = (g0, g1, g2, g3)
    ssems = (s0, s1, s2, s3)
    wid = lax.axis_index("s") * 2 + lax.axis_index("c")
    base = wid * GPW
    pltpu.sync_copy(idx3_hbm.at[wid], idxv)

    nj = GPW // 128
    gcp = [None] * nj
    scp = [None] * nj
    for j in range(nj):
        b = j % 4
        if j >= 4:
            scp[j - 4].wait()
        gcp[j] = pltpu.async_copy(emb_hbm.at[idxv.at[j]], bufs[b], gsems[b])
        if j >= 1:
            gcp[j - 1].wait()
            scp[j - 1] = pltpu.async_copy(
                bufs[(j - 1) % 4], out_hbm.at[pl.ds(base + (j - 1) * 128, 128)],
                ssems[(j - 1) % 4])
    gcp[nj - 1].wait()
    scp[nj - 1] = pltpu.async_copy(
        bufs[(nj - 1) % 4], out_hbm.at[pl.ds(base + (nj - 1) * 128, 128)],
        ssems[(nj - 1) % 4])
    for j in range(nj - 4, nj):
        scp[j].wait()


@functools.cache
def _get_stage3():
    return pl.kernel(
        _stage3_body,
        out_type=jax.ShapeDtypeStruct((NGATH, EMB), jnp.float32),
        mesh=plsc.VectorSubcoreMesh(core_axis_name="c", subcore_axis_name="s",
                                    num_cores=2, num_subcores=16),
        compiler_params=pltpu.CompilerParams(needs_layout_passes=False,
                                             use_tc_tiling_on_sc=False),
        scratch_types=(
            [pltpu.VMEM((GPW // 128, 128), jnp.int32)]
            + [pltpu.VMEM((128, EMB), jnp.float32)] * 4
            + [pltpu.SemaphoreType.DMA] * 8
        ),
    )


def _mm(x, w):
    return lax.dot_general(x, w, (((1,), (1,)), ((), ())),
                           preferred_element_type=jnp.float32)


def _pd(a, b):
    return jnp.sqrt(jnp.sum((a - b + EPS) ** 2, axis=1, keepdims=True))


def _stage4_body(*refs):
    (s0, s1, a0, a1, h0, h1, m0, m1) = refs[0:8]
    nsyn = refs[8:18]
    nhyp = refs[18:28]
    nmer = refs[28:38]
    (msk_s, msk_a, msk_h, msk_m) = refs[38:42]
    (wsyn, whyp, wmer, rh, bh, rm, bm) = refs[42:49]
    out_ref = refs[49]
    acc = refs[50]

    i = pl.program_id(0)

    @pl.when(i == 0)
    def _():
        for k in range(8):
            acc[k] = 0.0

    ws = wsyn[...]
    wh = whyp[...]
    wm = wmer[...]
    mh = lax.dot_general(rh[...], wh, (((1,), (0,)), ((), ())),
                         preferred_element_type=jnp.float32)
    mm_ = lax.dot_general(rm[...], wm, (((1,), (0,)), ((), ())),
                          preferred_element_type=jnp.float32)
    bhv = bh[...]
    bmv = bm[...]

    fm_s = 1.0 - (msk_s[...] == 0).astype(jnp.float32)
    fm_a = 1.0 - (msk_a[...] == 0).astype(jnp.float32)
    fm_h = 1.0 - (msk_h[...] == 0).astype(jnp.float32)
    fm_m = 1.0 - (msk_m[...] == 0).astype(jnp.float32)

    e1 = _mm(s0[...], ws)
    e2 = _mm(s1[...], ws)
    an = jnp.zeros_like(e1[:, :1])
    af = jnp.zeros_like(an)
    for j in range(NNEG):
        en = _mm(nsyn[j][...], ws)
        dn = _pd(e1, en)
        an = an + jnp.maximum(0.1 - dn, 0.0)
        af = af + jnp.maximum(dn - 1.5, 0.0)
    t_syn = (_pd(e1, e2) + an / NNEG + af / NNEG) * fm_s
    acc[0] = acc[0] + jnp.sum(t_syn)
    acc[1] = acc[1] + jnp.sum(fm_s)

    aa1 = _mm(a0[...], ws)
    aa2 = _mm(a1[...], ws)
    t_ant = jnp.maximum(1.0 - _pd(aa1, aa2), 0.0) * fm_a
    acc[2] = acc[2] + jnp.sum(t_ant)
    acc[3] = acc[3] + jnp.sum(fm_a)

    hh1 = _mm(h0[...], mh) + bhv
    hh2 = _mm(h1[...], wh)
    hn_acc = jnp.zeros_like(an)
    for j in range(NNEG):
        hn = _mm(nhyp[j][...], mh) + bhv
        hdn = _pd(hh2, hn)
        hn_acc = hn_acc + jnp.maximum(0.1 - hdn, 0.0)
    t_hyp = (_pd(hh1, hh2) + 3.0 * (hn_acc / NNEG)) * fm_h
    acc[4] = acc[4] + jnp.sum(t_hyp)
    acc[5] = acc[5] + jnp.sum(fm_h)

    mm1 = _mm(m0[...], mm_) + bmv
    mm2 = _mm(m1[...], wm)
    mn_acc = jnp.zeros_like(an)
    for j in range(NNEG):
        mn = _mm(nmer[j][...], mm_) + bmv
        mdn = _pd(mm2, mn)
        mn_acc = mn_acc + jnp.maximum(0.1 - mdn, 0.0)
    t_mer = (_pd(mm1, mm2) + mn_acc / NNEG) * fm_m
    acc[6] = acc[6] + jnp.sum(t_mer)
    acc[7] = acc[7] + jnp.sum(fm_m)

    @pl.when(i == pl.num_programs(0) - 1)
    def _():
        l0 = acc[0] / jnp.maximum(acc[1], 1.0)
        l1 = acc[2] / jnp.maximum(acc[3], 1.0)
        l2 = acc[4] / jnp.maximum(acc[5], 1.0)
        l3 = acc[6] / jnp.maximum(acc[7], 1.0)
        lane = lax.broadcasted_iota(jnp.int32, (1, 128), 1)
        outv = (jnp.where(lane == 0, l0, 0.0) + jnp.where(lane == 1, l1, 0.0)
                + jnp.where(lane == 2, l2, 0.0) + jnp.where(lane == 3, l3, 0.0))
        out_ref[...] = outv.astype(jnp.float32)


def _build_stage4(interpret=False):
    bs = 1024
    grid = (B // bs,)
    gspec = lambda c: pl.BlockSpec((bs, EMB), lambda i, c=c: (c * (B // bs) + i, 0))
    in_specs = []
    for c in range(8):
        in_specs.append(gspec(c))
    for r in range(3):
        for j in range(NNEG):
            off = (NPAIR + r * B * NNEG + j * B) // bs
            in_specs.append(pl.BlockSpec((bs, EMB), lambda i, off=off: (off + i, 0)))
    for _ in range(4):
        in_specs.append(pl.BlockSpec((bs, 1), lambda i: (i, 0)))
    for _ in range(3):
        in_specs.append(pl.BlockSpec((WN, EMB), lambda i: (0, 0)))
    in_specs.append(pl.BlockSpec((WN, WN), lambda i: (0, 0)))
    in_specs.append(pl.BlockSpec((1, WN), lambda i: (0, 0)))
    in_specs.append(pl.BlockSpec((WN, WN), lambda i: (0, 0)))
    in_specs.append(pl.BlockSpec((1, WN), lambda i: (0, 0)))
    return pl.pallas_call(
        _stage4_body,
        grid=grid,
        in_specs=in_specs,
        out_specs=pl.BlockSpec((1, 128), lambda i: (0, 0)),
        out_shape=jax.ShapeDtypeStruct((1, 128), jnp.float32),
        scratch_shapes=[pltpu.SMEM((8,), jnp.float32)],
        compiler_params=pltpu.CompilerParams(vmem_limit_bytes=100 * 1024 * 1024),
        interpret=interpret,
    )


_stage4 = _build_stage4()


def kernel(synonyms, antonyms, hypernyms, meronyms, emb_table, vocab_freq,
           syn_proj_w, hypn_proj_w, mern_proj_w,
           hypn_rel_w, hypn_rel_b, mern_rel_w, mern_rel_b):
    skey = jax.random.key(42)
    us = [jax.random.uniform(jax.random.fold_in(skey, i), (B, NNEG), jnp.float32)
          for i in (1, 2, 3)]
    u_all = jnp.concatenate([u.T.reshape(-1) for u in us]).reshape(3840, 128)

    f_pad = jnp.concatenate(
        [vocab_freq, jnp.zeros((VP - V,), jnp.float32)]).reshape(64, 128, 128)

    p3, c3, r2 = _stage1(f_pad, u_all)
    p2d = p3.reshape(NROW32, 32)
    c1 = c3.reshape(NROW32)
    r1 = r2.reshape(NQ)

    nidx = _get_stage2()(p2d, c1, r1)

    pairs = jnp.stack([synonyms, antonyms, hypernyms, meronyms])
    pairs = pairs.transpose(0, 2, 1).reshape(-1).astype(jnp.int32)
    idx3 = jnp.concatenate([pairs, nidx]).reshape(NW, GPW // 128, 128)

    g = _get_stage3()(idx3, emb_table)

    gargs = [g] * 38
    margs = [synonyms[:, :1], antonyms[:, :1], hypernyms[:, :1], meronyms[:, :1]]
    wargs = [syn_proj_w, hypn_proj_w, mern_proj_w,
             hypn_rel_w, hypn_rel_b.reshape(1, WN),
             mern_rel_w, mern_rel_b.reshape(1, WN)]
    out2d = _stage4(*gargs, *margs, *wargs)
    return out2d[0, :4]


# speedup vs baseline: 8.3840x; 1.0007x over previous
import functools

import jax
import jax.numpy as jnp
from jax import lax
from jax.experimental import pallas as pl
from jax.experimental.pallas import tpu as pltpu
from jax.experimental.pallas import tpu_sc as plsc

V = 1000000
VP = 1 << 20
EMB = 64
WN = 64
B = 16384
NNEG = 10
NQ = 3 * B * NNEG
EPS = 1e-6

NROW32 = VP // 32
NW = 32
QPW = NQ // NW
CQ = 1536
NCH = QPW // CQ
NPAIR = 8 * B
NGATH = NPAIR + NQ
GPW = NGATH // NW


def _stage1_body(f_ref, u_ref, p_ref, c_ref, r_ref, carry_ref):
    i = pl.program_id(0)

    @pl.when(i == 0)
    def _():
        carry_ref[0] = 0.0

    x = f_ref[0]
    g = jnp.where(x > 0.0, jnp.exp(0.75 * jnp.log(jnp.where(x > 0.0, x, 1.0))), 0.0)

    ri = lax.broadcasted_iota(jnp.int32, (128, 128), 0)
    ci = lax.broadcasted_iota(jnp.int32, (128, 128), 1)
    upper = (ri <= ci).astype(jnp.float32)
    lstrict = (ri > ci).astype(jnp.float32)

    pref = lax.dot_general(g, upper, (((1,), (0,)), ((), ())),
                           precision=lax.Precision.HIGHEST,
                           preferred_element_type=jnp.float32)
    rowsum = jnp.sum(g, axis=1, keepdims=True)
    roff = lax.dot_general(lstrict, rowsum, (((1,), (0,)), ((), ())),
                           precision=lax.Precision.HIGHEST,
                           preferred_element_type=jnp.float32)
    carry = carry_ref[0]
    p = pref + roff + carry
    p_ref[0] = p

    r8 = lax.broadcasted_iota(jnp.int32, (128, 4), 0)
    c8 = lax.broadcasted_iota(jnp.int32, (128, 4), 1)
    esel = (r8 == 32 * c8 + 31).astype(jnp.float32)
    c_ref[0] = lax.dot_general(p, esel, (((1,), (0,)), ((), ())),
                               precision=lax.Precision.HIGHEST,
                               preferred_element_type=jnp.float32)

    carry_ref[0] = carry + jnp.sum(g)

    @pl.when(i == 63)
    def _():
        t = carry_ref[0]
        r_ref[...] = t * (1.0 - u_ref[...])


def _build_stage1(interpret=False):
    return pl.pallas_call(
        _stage1_body,
        grid=(64,),
        in_specs=[
            pl.BlockSpec((1, 128, 128), lambda i: (i, 0, 0)),
            pl.BlockSpec((3840, 128), lambda i: (0, 0)),
        ],
        out_specs=[
            pl.BlockSpec((1, 128, 128), lambda i: (i, 0, 0)),
            pl.BlockSpec((1, 128, 4), lambda i: (i, 0, 0)),
            pl.BlockSpec((3840, 128), lambda i: (0, 0)),
        ],
        out_shape=[
            jax.ShapeDtypeStruct((64, 128, 128), jnp.float32),
            jax.ShapeDtypeStruct((64, 128, 4), jnp.float32),
            jax.ShapeDtypeStruct((3840, 128), jnp.float32),
        ],
        scratch_shapes=[pltpu.SMEM((1,), jnp.float32)],
        interpret=interpret,
    )


_stage1 = _build_stage1()


def _stage2_body(p2d_hbm, c_hbm, r_hbm, out_hbm, cv, qall, rowv2, rowsbuf, outall, sem):
    wid = lax.axis_index("s") * 2 + lax.axis_index("c")
    base = wid * QPW
    pltpu.sync_copy(c_hbm, cv)
    pltpu.sync_copy(r_hbm.at[pl.ds(base, QPW)], qall)

    def chunk_body(ch, _):
        coff = ch * CQ

        def coarse_body(j2, _):
            for k in range(8):
                q = qall[pl.ds(coff + j2 * 128 + k * 16, 16)]
                lo = jnp.zeros((16,), jnp.int32)
                for st in range(15):
                    h = 1 << (14 - st)
                    probe = lo + (h - 1)
                    val = plsc.load_gather(cv, [probe])
                    lo = jnp.where(val < q, lo + h, lo)
                rowv2[j2, pl.ds(k * 16, 16)] = lo
            return 0

        lax.fori_loop(0, 12, coarse_body, 0)

        cps = [
            pltpu.async_copy(p2d_hbm.at[rowv2.at[j]], rowsbuf.at[pl.ds(j * 128, 128)], sem)
            for j in range(12)
        ]
        for cp in cps:
            cp.wait()

        def refine_body(j2, _):
            for k in range(8):
                off = j2 * 128 + k * 16
                q = qall[pl.ds(coff + off, 16)]
                lo = rowv2[j2, pl.ds(k * 16, 16)]
                lanesel = lax.iota(jnp.int32, 16) + off
                pos = jnp.zeros((16,), jnp.int32)
                for st in range(5):
                    h = 1 << (4 - st)
                    probe = pos + (h - 1)
                    vals = plsc.load_gather(rowsbuf, [lanesel, probe])
                    pos = jnp.where(vals < q, pos + h, pos)
                idx = jnp.minimum(lo * 32 + pos, V - 1)
                outall[pl.ds(coff + off, 16)] = idx
            return 0

        lax.fori_loop(0, 12, refine_body, 0)
        return 0

    lax.fori_loop(0, NCH, chunk_body, 0)
    pltpu.sync_copy(outall, out_hbm.at[pl.ds(base, QPW)])


@functools.cache
def _get_stage2():
    return pl.kernel(
        _stage2_body,
        out_type=jax.ShapeDtypeStruct((NQ,), jnp.int32),
        mesh=plsc.VectorSubcoreMesh(core_axis_name="c", subcore_axis_name="s",
                                    num_cores=2, num_subcores=16),
        compiler_params=pltpu.CompilerParams(needs_layout_passes=False, use_tc_tiling_on_sc=False),
        scratch_types=[
            pltpu.VMEM((NROW32,), jnp.float32),
            pltpu.VMEM((QPW,), jnp.float32),
            pltpu.VMEM((12, 128), jnp.int32),
            pltpu.VMEM((CQ, 32), jnp.float32),
            pltpu.VMEM((QPW,), jnp.int32),
            pltpu.SemaphoreType.DMA,
        ],
    )


def _stage3_body(idx3_hbm, emb_hbm, out_hbm, idxv, b0, b1, b2, b3,
                 g0, g1, g2, g3, s0, s1, s2, s3):
    bufs = (b0, b1, b2, b3)
    gsems = (g0, g1, g2, g3)
    ssems = (s0, s1, s2, s3)
    wid = lax.axis_index("s") * 2 + lax.axis_index("c")
    base = wid * GPW
    pltpu.sync_copy(idx3_hbm.at[wid], idxv)

    nj = GPW // 128
    gcp = [None] * nj
    scp = [None] * nj
    for j in range(nj):
        b = j % 4
        if j >= 4:
            scp[j - 4].wait()
        gcp[j] = pltpu.async_copy(emb_hbm.at[idxv.at[j]], bufs[b], gsems[b])
        if j >= 1:
            gcp[j - 1].wait()
            scp[j - 1] = pltpu.async_copy(
                bufs[(j - 1) % 4], out_hbm.at[pl.ds(base + (j - 1) * 128, 128)],
                ssems[(j - 1) % 4])
    gcp[nj - 1].wait()
    scp[nj - 1] = pltpu.async_copy(
        bufs[(nj - 1) % 4], out_hbm.at[pl.ds(base + (nj - 1) * 128, 128)],
        ssems[(nj - 1) % 4])
    for j in range(nj - 4, nj):
        scp[j].wait()


@functools.cache
def _get_stage3():
    return pl.kernel(
        _stage3_body,
        out_type=jax.ShapeDtypeStruct((NGATH, EMB), jnp.float32),
        mesh=plsc.VectorSubcoreMesh(core_axis_name="c", subcore_axis_name="s",
                                    num_cores=2, num_subcores=16),
        compiler_params=pltpu.CompilerParams(needs_layout_passes=False,
                                             use_tc_tiling_on_sc=False),
        scratch_types=(
            [pltpu.VMEM((GPW // 128, 128), jnp.int32)]
            + [pltpu.VMEM((128, EMB), jnp.float32)] * 4
            + [pltpu.SemaphoreType.DMA] * 8
        ),
    )


def _mm(x, w):
    return lax.dot_general(x, w, (((1,), (1,)), ((), ())),
                           preferred_element_type=jnp.float32)


def _pd(a, b):
    return jnp.sqrt(jnp.sum((a - b + EPS) ** 2, axis=1, keepdims=True))


def _stage4_body(*refs):
    (s0, s1, a0, a1, h0, h1, m0, m1) = refs[0:8]
    nsyn = refs[8:18]
    nhyp = refs[18:28]
    nmer = refs[28:38]
    (msk_s, msk_a, msk_h, msk_m) = refs[38:42]
    (wsyn, whyp, wmer, rh, bh, rm, bm) = refs[42:49]
    out_ref = refs[49]
    acc = refs[50]

    i = pl.program_id(0)

    @pl.when(i == 0)
    def _():
        for k in range(8):
            acc[k] = 0.0

    ws = wsyn[...]
    wh = whyp[...]
    wm = wmer[...]
    mh = lax.dot_general(rh[...], wh, (((1,), (0,)), ((), ())),
                         preferred_element_type=jnp.float32)
    mm_ = lax.dot_general(rm[...], wm, (((1,), (0,)), ((), ())),
                          preferred_element_type=jnp.float32)
    bhv = bh[...]
    bmv = bm[...]

    fm_s = 1.0 - (msk_s[...] == 0).astype(jnp.float32)
    fm_a = 1.0 - (msk_a[...] == 0).astype(jnp.float32)
    fm_h = 1.0 - (msk_h[...] == 0).astype(jnp.float32)
    fm_m = 1.0 - (msk_m[...] == 0).astype(jnp.float32)

    e1 = _mm(s0[...], ws)
    e2 = _mm(s1[...], ws)
    an = jnp.zeros_like(e1[:, :1])
    af = jnp.zeros_like(an)
    for j in range(NNEG):
        en = _mm(nsyn[j][...], ws)
        dn = _pd(e1, en)
        an = an + jnp.maximum(0.1 - dn, 0.0)
        af = af + jnp.maximum(dn - 1.5, 0.0)
    t_syn = (_pd(e1, e2) + an / NNEG + af / NNEG) * fm_s
    acc[0] = acc[0] + jnp.sum(t_syn)
    acc[1] = acc[1] + jnp.sum(fm_s)

    aa1 = _mm(a0[...], ws)
    aa2 = _mm(a1[...], ws)
    t_ant = jnp.maximum(1.0 - _pd(aa1, aa2), 0.0) * fm_a
    acc[2] = acc[2] + jnp.sum(t_ant)
    acc[3] = acc[3] + jnp.sum(fm_a)

    hh1 = _mm(h0[...], mh) + bhv
    hh2 = _mm(h1[...], wh)
    hn_acc = jnp.zeros_like(an)
    for j in range(NNEG):
        hn = _mm(nhyp[j][...], mh) + bhv
        hdn = _pd(hh2, hn)
        hn_acc = hn_acc + jnp.maximum(0.1 - hdn, 0.0)
    t_hyp = (_pd(hh1, hh2) + 3.0 * (hn_acc / NNEG)) * fm_h
    acc[4] = acc[4] + jnp.sum(t_hyp)
    acc[5] = acc[5] + jnp.sum(fm_h)

    mm1 = _mm(m0[...], mm_) + bmv
    mm2 = _mm(m1[...], wm)
    mn_acc = jnp.zeros_like(an)
    for j in range(NNEG):
        mn = _mm(nmer[j][...], mm_) + bmv
        mdn = _pd(mm2, mn)
        mn_acc = mn_acc + jnp.maximum(0.1 - mdn, 0.0)
    t_mer = (_pd(mm1, mm2) + mn_acc / NNEG) * fm_m
    acc[6] = acc[6] + jnp.sum(t_mer)
    acc[7] = acc[7] + jnp.sum(fm_m)

    @pl.when(i == pl.num_programs(0) - 1)
    def _():
        l0 = acc[0] / jnp.maximum(acc[1], 1.0)
        l1 = acc[2] / jnp.maximum(acc[3], 1.0)
        l2 = acc[4] / jnp.maximum(acc[5], 1.0)
        l3 = acc[6] / jnp.maximum(acc[7], 1.0)
        lane = lax.broadcasted_iota(jnp.int32, (1, 128), 1)
        outv = (jnp.where(lane == 0, l0, 0.0) + jnp.where(lane == 1, l1, 0.0)
                + jnp.where(lane == 2, l2, 0.0) + jnp.where(lane == 3, l3, 0.0))
        out_ref[...] = outv.astype(jnp.float32)


def _build_stage4(interpret=False):
    bs = 1024
    grid = (B // bs,)
    gspec = lambda c: pl.BlockSpec((bs, EMB), lambda i, c=c: (c * (B // bs) + i, 0))
    in_specs = []
    for c in range(8):
        in_specs.append(gspec(c))
    for r in range(3):
        for j in range(NNEG):
            off = (NPAIR + r * B * NNEG + j * B) // bs
            in_specs.append(pl.BlockSpec((bs, EMB), lambda i, off=off: (off + i, 0)))
    for _ in range(4):
        in_specs.append(pl.BlockSpec((bs, 1), lambda i: (i, 0)))
    for _ in range(3):
        in_specs.append(pl.BlockSpec((WN, EMB), lambda i: (0, 0)))
    in_specs.append(pl.BlockSpec((WN, WN), lambda i: (0, 0)))
    in_specs.append(pl.BlockSpec((1, WN), lambda i: (0, 0)))
    in_specs.append(pl.BlockSpec((WN, WN), lambda i: (0, 0)))
    in_specs.append(pl.BlockSpec((1, WN), lambda i: (0, 0)))
    return pl.pallas_call(
        _stage4_body,
        grid=grid,
        in_specs=in_specs,
        out_specs=pl.BlockSpec((1, 128), lambda i: (0, 0)),
        out_shape=jax.ShapeDtypeStruct((1, 128), jnp.float32),
        scratch_shapes=[pltpu.SMEM((8,), jnp.float32)],
        compiler_params=pltpu.CompilerParams(vmem_limit_bytes=100 * 1024 * 1024),
        interpret=interpret,
    )


_stage4 = _build_stage4()


def kernel(synonyms, antonyms, hypernyms, meronyms, emb_table, vocab_freq,
           syn_proj_w, hypn_proj_w, mern_proj_w,
           hypn_rel_w, hypn_rel_b, mern_rel_w, mern_rel_b):
    skey = jax.random.key(42)
    us = [jax.random.uniform(jax.random.fold_in(skey, i), (B, NNEG), jnp.float32)
          for i in (1, 2, 3)]
    u_all = jnp.concatenate([u.T.reshape(-1) for u in us]).reshape(3840, 128)

    f_pad = jnp.concatenate(
        [vocab_freq, jnp.zeros((VP - V,), jnp.float32)]).reshape(64, 128, 128)

    p3, c3, r2 = _stage1(f_pad, u_all)
    p2d = p3.reshape(NROW32, 32)
    c1 = c3.reshape(NROW32)
    r1 = r2.reshape(NQ)

    nidx = _get_stage2()(p2d, c1, r1)

    pairs = jnp.stack([synonyms, antonyms, hypernyms, meronyms])
    pairs = pairs.transpose(0, 2, 1).reshape(-1).astype(jnp.int32)
    idx3 = jnp.concatenate([pairs, nidx]).reshape(NW, GPW // 128, 128)

    g = _get_stage3()(idx3, emb_table)

    gargs = [g] * 38
    margs = [synonyms[:, :1], antonyms[:, :1], hypernyms[:, :1], meronyms[:, :1]]
    wargs = [syn_proj_w, hypn_proj_w, mern_proj_w,
             hypn_rel_w, hypn_rel_b.reshape(1, WN),
             mern_rel_w, mern_rel_b.reshape(1, WN)]
    out2d = _stage4(*gargs, *margs, *wargs)
    return out2d[0, :4]
